# Initial kernel scaffold; baseline (speedup 1.0000x reference)
import jax
import jax.numpy as jnp
from jax.experimental import pallas as pl


def kernel(v1, v2, memory_v1, memory_v2, y, idx):
    raise NotImplementedError("write your pallas kernel here")


# speedup vs baseline: 9.0508x; 9.0508x over previous
import functools

import jax
import jax.numpy as jnp
from jax import lax
from jax.experimental import pallas as pl
from jax.experimental.pallas import tpu as pltpu
from jax.experimental.pallas import tpu_sc as plsc

B = 1024
D = 128
N = 100000
K1 = 1024
T = 0.07
MOM = 0.5

NC = 2
NS = 16
NW = NC * NS
BPW = B // NW
RCH = 128
NCH = K1 // RCH
LANES = 16


def _hsum_all(s, lane):
    for k in (8, 4, 2, 1):
        perm = lane ^ k
        s = s + s.at[perm].get(mode="promise_in_bounds")
    return s


def _sc_phase1(v1_hbm, v2_hbm, m1_hbm, m2_hbm, y_hbm, idx_hbm,
               eA_hbm, eB_hbm, sums_hbm, my1_hbm, my2_hbm,
               idx_v, va_v, vb_v, rows_v, dots_v, yv_v, myrows_v, sums_v,
               gsem0, gsem1, ysem):
    wid = lax.axis_index("s") * NC + lax.axis_index("c")
    b0 = wid * BPW
    lane = lax.iota(jnp.int32, LANES)

    pltpu.sync_copy(y_hbm.at[pl.ds(b0, BPW)], yv_v)
    pltpu.async_copy(m1_hbm.at[yv_v], myrows_v, ysem).wait()
    pltpu.sync_copy(myrows_v, my1_hbm.at[pl.ds(b0, BPW)])
    pltpu.async_copy(m2_hbm.at[yv_v], myrows_v, ysem).wait()
    pltpu.sync_copy(myrows_v, my2_hbm.at[pl.ds(b0, BPW)])

    def _start(m_hbm, c, slot, sem):
        pltpu.make_async_copy(m_hbm.at[idx_v.at[c]], rows_v.at[slot], sem).start()

    def _wait(m_hbm, c, slot, sem):
        pltpu.make_async_copy(m_hbm.at[idx_v.at[c]], rows_v.at[slot], sem).wait()

    def _compute_chunk(slot, c, vv, sv):
        def g_body(g, sv2):
            accv = jnp.zeros((LANES,), jnp.float32)
            for u in range(LANES):
                r = g * LANES + u
                s = rows_v[slot, r, pl.ds(0, LANES)] * vv[0]
                for i in range(1, D // LANES):
                    s = s + rows_v[slot, r, pl.ds(i * LANES, LANES)] * vv[i]
                s = _hsum_all(s, lane)
                accv = jnp.where(lane == u, s, accv)
            e = jnp.exp(accv / T)
            dots_v[pl.ds(c * RCH + g * LANES, LANES)] = e
            return sv2 + e
        return lax.fori_loop(0, RCH // LANES, g_body, sv)

    def _table_pass(m_hbm, v_ref, out_hbm, b, sumvec):
        vv = [v_ref[pl.ds(i * LANES, LANES)] for i in range(D // LANES)]
        _start(m_hbm, 0, 0, gsem0)

        def pair_body(p, sv):
            c0 = 2 * p
            _start(m_hbm, c0 + 1, 1, gsem1)
            _wait(m_hbm, c0, 0, gsem0)
            sv = _compute_chunk(0, c0, vv, sv)

            @pl.when(p + 1 < NCH // 2)
            def _():
                _start(m_hbm, c0 + 2, 0, gsem0)
            _wait(m_hbm, c0 + 1, 1, gsem1)
            sv = _compute_chunk(1, c0 + 1, vv, sv)
            return sv

        sumvec = lax.fori_loop(0, NCH // 2, pair_body, sumvec)
        pltpu.sync_copy(dots_v, out_hbm.at[b])
        return sumvec

    def b_body(j, carry):
        svA, svB = carry
        b = b0 + j
        pltpu.sync_copy(idx_hbm.at[b], idx_v)
        pltpu.sync_copy(v2_hbm.at[b], va_v)
        pltpu.sync_copy(v1_hbm.at[b], vb_v)
        svA = _table_pass(m1_hbm, va_v, eA_hbm, b, svA)
        svB = _table_pass(m2_hbm, vb_v, eB_hbm, b, svB)
        return (svA, svB)

    z16 = jnp.zeros((LANES,), jnp.float32)
    svA, svB = lax.fori_loop(0, BPW, b_body, (z16, z16))

    sA = _hsum_all(svA, lane)
    sB = _hsum_all(svB, lane)
    sums_v[...] = jnp.where(lane == 0, sA, jnp.where(lane == 1, sB, 0.0))
    pltpu.sync_copy(sums_v, sums_hbm.at[wid])


_phase1 = functools.partial(
    pl.kernel,
    out_type=[
        jax.ShapeDtypeStruct((B, K1), jnp.float32),
        jax.ShapeDtypeStruct((B, K1), jnp.float32),
        jax.ShapeDtypeStruct((NW, LANES), jnp.float32),
        jax.ShapeDtypeStruct((B, D), jnp.float32),
        jax.ShapeDtypeStruct((B, D), jnp.float32),
    ],
    mesh=plsc.VectorSubcoreMesh(core_axis_name="c", subcore_axis_name="s",
                                num_cores=NC, num_subcores=NS),
    scratch_types=[
        pltpu.VMEM((NCH, RCH), jnp.int32),
        pltpu.VMEM((D,), jnp.float32),
        pltpu.VMEM((D,), jnp.float32),
        pltpu.VMEM((2, RCH, D), jnp.float32),
        pltpu.VMEM((K1,), jnp.float32),
        pltpu.VMEM((BPW,), jnp.int32),
        pltpu.VMEM((BPW, D), jnp.float32),
        pltpu.VMEM((LANES,), jnp.float32),
        pltpu.SemaphoreType.DMA,
        pltpu.SemaphoreType.DMA,
        pltpu.SemaphoreType.DMA,
    ],
)(_sc_phase1)


CH = 128


def _tc_phase2_body(sums_ref, eA_ref, eB_ref, my1_ref, my2_ref, v1_ref, v2_ref,
                    outA_ref, outB_ref, upd1_ref, upd2_ref):
    s = sums_ref[...]
    scale = jnp.float32(N) / jnp.float32(B * K1)
    zA = jnp.sum(s[:, 0]) * scale
    zB = jnp.sum(s[:, 1]) * scale
    outA_ref[...] = eA_ref[...] / zA
    outB_ref[...] = eB_ref[...] / zB

    @pl.when(pl.program_id(0) == 0)
    def _():
        l1 = my1_ref[...] * MOM + v1_ref[...] * (1.0 - MOM)
        n1 = jnp.sqrt(jnp.sum(l1 * l1, axis=1, keepdims=True))
        upd1_ref[...] = l1 / n1
        l2 = my2_ref[...] * MOM + v2_ref[...] * (1.0 - MOM)
        n2 = jnp.sqrt(jnp.sum(l2 * l2, axis=1, keepdims=True))
        upd2_ref[...] = l2 / n2


_phase2 = pl.pallas_call(
    _tc_phase2_body,
    grid=(B // CH,),
    in_specs=[
        pl.BlockSpec((NW, LANES), lambda i: (0, 0)),
        pl.BlockSpec((CH, K1), lambda i: (i, 0)),
        pl.BlockSpec((CH, K1), lambda i: (i, 0)),
        pl.BlockSpec((B, D), lambda i: (0, 0)),
        pl.BlockSpec((B, D), lambda i: (0, 0)),
        pl.BlockSpec((B, D), lambda i: (0, 0)),
        pl.BlockSpec((B, D), lambda i: (0, 0)),
    ],
    out_specs=[
        pl.BlockSpec((CH, K1), lambda i: (i, 0)),
        pl.BlockSpec((CH, K1), lambda i: (i, 0)),
        pl.BlockSpec((B, D), lambda i: (0, 0)),
        pl.BlockSpec((B, D), lambda i: (0, 0)),
    ],
    out_shape=[
        jax.ShapeDtypeStruct((B, K1), jnp.float32),
        jax.ShapeDtypeStruct((B, K1), jnp.float32),
        jax.ShapeDtypeStruct((B, D), jnp.float32),
        jax.ShapeDtypeStruct((B, D), jnp.float32),
    ],
)


def _tc_phase3_body(upd1_ref, upd2_ref, y_ref, win_ref, m1_ref, m2_ref,
                    o1_ref, o2_ref, sem1, sem2):
    def issue(i, _):
        wi = win_ref[i]
        yi = y_ref[i]
        pltpu.make_async_copy(upd1_ref.at[pl.ds(wi, 1)],
                              o1_ref.at[pl.ds(yi, 1)], sem1).start()
        pltpu.make_async_copy(upd2_ref.at[pl.ds(wi, 1)],
                              o2_ref.at[pl.ds(yi, 1)], sem2).start()
        return 0

    lax.fori_loop(0, B, issue, 0)
    pltpu.make_async_copy(upd1_ref, o1_ref.at[pl.ds(0, B)], sem1).wait()
    pltpu.make_async_copy(upd2_ref, o2_ref.at[pl.ds(0, B)], sem2).wait()


_phase3 = pl.pallas_call(
    _tc_phase3_body,
    in_specs=[
        pl.BlockSpec(memory_space=pltpu.VMEM),
        pl.BlockSpec(memory_space=pltpu.VMEM),
        pl.BlockSpec(memory_space=pltpu.SMEM),
        pl.BlockSpec(memory_space=pltpu.SMEM),
        pl.BlockSpec(memory_space=pltpu.MemorySpace.HBM),
        pl.BlockSpec(memory_space=pltpu.MemorySpace.HBM),
    ],
    out_specs=[
        pl.BlockSpec(memory_space=pltpu.MemorySpace.HBM),
        pl.BlockSpec(memory_space=pltpu.MemorySpace.HBM),
    ],
    out_shape=[
        jax.ShapeDtypeStruct((N, D), jnp.float32),
        jax.ShapeDtypeStruct((N, D), jnp.float32),
    ],
    input_output_aliases={4: 0, 5: 1},
    scratch_shapes=[pltpu.SemaphoreType.DMA, pltpu.SemaphoreType.DMA],
)


def kernel(v1, v2, memory_v1, memory_v2, y, idx):
    y = y.astype(jnp.int32)
    idx3 = idx.astype(jnp.int32).reshape(B, NCH, RCH)

    eA, eB, sums, my1, my2 = _phase1(
        v1, v2, memory_v1, memory_v2, y, idx3)

    outA, outB, upd1, upd2 = _phase2(sums, eA, eB, my1, my2, v1, v2)

    bi = jnp.arange(B, dtype=jnp.int32)
    winner = jnp.max(jnp.where(y[None, :] == y[:, None], bi[None, :], -1),
                     axis=1).astype(jnp.int32)

    new_m1, new_m2 = _phase3(upd1, upd2, y, winner, memory_v1, memory_v2)

    out_v1 = outB[..., None]
    out_v2 = outA[..., None]
    return (out_v1, out_v2, new_m1, new_m2)


# speedup vs baseline: 10.1916x; 1.1260x over previous
import functools

import jax
import jax.numpy as jnp
from jax import lax
from jax.experimental import pallas as pl
from jax.experimental.pallas import tpu as pltpu
from jax.experimental.pallas import tpu_sc as plsc

B = 1024
D = 128
N = 100000
K1 = 1024
T = 0.07
MOM = 0.5

NC = 2
NS = 16
NW = NC * NS
BPW = B // NW
RCH = 128
NCH = K1 // RCH
LANES = 16


def _hsum_all(s, lane):
    for k in (8, 4, 2, 1):
        perm = lane ^ k
        s = s + s.at[perm].get(mode="promise_in_bounds")
    return s


def _bf16_fast(x):
    bits = lax.bitcast_convert_type(x, jnp.int32)
    r = jnp.bitwise_and(bits + jnp.int32(0x8000), jnp.int32(-65536))
    return lax.bitcast_convert_type(r, jnp.float32)


def _bf16_rtne(x):
    bits = lax.bitcast_convert_type(x, jnp.int32)
    odd = jnp.bitwise_and(lax.shift_right_logical(bits, 16), jnp.int32(1))
    r = jnp.bitwise_and(bits + jnp.int32(0x7FFF) + odd, jnp.int32(-65536))
    return lax.bitcast_convert_type(r, jnp.float32)


def _sc_phase1(v1_hbm, v2_hbm, m1_hbm, m2_hbm, y_hbm, idx_hbm,
               eA_hbm, eB_hbm, my1_hbm, my2_hbm,
               idx_v, va_v, vb_v, rows_v, dots_v, yv_v, myrows_v,
               gsem0, gsem1, psem, dsem0, dsem1, ysem):
    wid = lax.axis_index("s") * NC + lax.axis_index("c")
    b0 = wid * BPW
    lane = lax.iota(jnp.int32, LANES)

    pltpu.sync_copy(y_hbm.at[pl.ds(b0, BPW)], yv_v)
    pltpu.make_async_copy(m1_hbm.at[yv_v], myrows_v.at[0], ysem).start()
    pltpu.make_async_copy(m2_hbm.at[yv_v], myrows_v.at[1], ysem).start()

    def _start(m_hbm, par, c, slot, sem):
        pltpu.make_async_copy(m_hbm.at[idx_v.at[par, c]], rows_v.at[slot],
                              sem).start()

    def _wait(m_hbm, par, c, slot, sem):
        pltpu.make_async_copy(m_hbm.at[idx_v.at[par, c]], rows_v.at[slot],
                              sem).wait()

    def _start_pf(par, b):
        pltpu.make_async_copy(idx_hbm.at[b], idx_v.at[par], psem).start()
        pltpu.make_async_copy(v2_hbm.at[b], va_v.at[par], psem).start()
        pltpu.make_async_copy(v1_hbm.at[b], vb_v.at[par], psem).start()

    def _wait_pf(par, b):
        pltpu.make_async_copy(idx_hbm.at[b], idx_v.at[par], psem).wait()
        pltpu.make_async_copy(v2_hbm.at[b], va_v.at[par], psem).wait()
        pltpu.make_async_copy(v1_hbm.at[b], vb_v.at[par], psem).wait()

    def _compute_chunk(slot, dslot, c, vv, sv):
        def g_body(g, sv2):
            accv = jnp.zeros((LANES,), jnp.float32)
            for u in range(LANES):
                r = g * LANES + u
                s = _bf16_fast(rows_v[slot, r, pl.ds(0, LANES)]) * vv[0]
                for i in range(1, D // LANES):
                    s = s + (_bf16_fast(rows_v[slot, r, pl.ds(i * LANES, LANES)])
                             * vv[i])
                s = _hsum_all(s, lane)
                accv = jnp.where(lane == u, s, accv)
            dots_v[dslot, pl.ds(c * RCH + g * LANES, LANES)] = accv
            return sv2
        return lax.fori_loop(0, RCH // LANES, g_body, sv)

    def _table_pass(m_hbm, par, dslot, dsem, out_hbm, b, j, sumvec, prefetch):
        @pl.when(j > 0)
        def _():
            pltpu.make_async_copy(dots_v.at[dslot], out_hbm.at[b], dsem).wait()

        vv = [_bf16_rtne(va_v[par, pl.ds(i * LANES, LANES)]) if dslot == 0
              else _bf16_rtne(vb_v[par, pl.ds(i * LANES, LANES)])
              for i in range(D // LANES)]

        def pair_body(p, sv):
            c0 = 2 * p
            _start(m_hbm, par, c0 + 1, 1, gsem1)
            _wait(m_hbm, par, c0, 0, gsem0)
            sv = _compute_chunk(0, dslot, c0, vv, sv)

            @pl.when(p + 1 < NCH // 2)
            def _():
                _start(m_hbm, par, c0 + 2, 0, gsem0)

            @pl.when(p + 1 == NCH // 2)
            def _():
                prefetch()
            _wait(m_hbm, par, c0 + 1, 1, gsem1)
            sv = _compute_chunk(1, dslot, c0 + 1, vv, sv)
            return sv

        sumvec = lax.fori_loop(0, NCH // 2, pair_body, sumvec)
        pltpu.make_async_copy(dots_v.at[dslot], out_hbm.at[b], dsem).start()
        return sumvec

    def b_body(j, carry):
        par = jnp.bitwise_and(j, 1)
        npar = 1 - par
        b = b0 + j

        @pl.when(j + 1 < BPW)
        def _():
            _start_pf(npar, b + 1)

        def pf_tableB():
            _start(m2_hbm, par, 0, 0, gsem0)

        def pf_next_tableA():
            @pl.when(j + 1 < BPW)
            def _():
                _wait_pf(npar, b + 1)
                _start(m1_hbm, npar, 0, 0, gsem0)

        _table_pass(m1_hbm, par, 0, dsem0, eA_hbm, b, j, 0, pf_tableB)
        _table_pass(m2_hbm, par, 1, dsem1, eB_hbm, b, j, 0, pf_next_tableA)
        return carry

    _start_pf(0, b0)
    _wait_pf(0, b0)
    _start(m1_hbm, 0, 0, 0, gsem0)

    lax.fori_loop(0, BPW, b_body, 0)

    bl = b0 + BPW - 1
    pltpu.make_async_copy(dots_v.at[0], eA_hbm.at[bl], dsem0).wait()
    pltpu.make_async_copy(dots_v.at[1], eB_hbm.at[bl], dsem1).wait()

    pltpu.make_async_copy(m1_hbm.at[yv_v], myrows_v.at[0], ysem).wait()
    pltpu.make_async_copy(m2_hbm.at[yv_v], myrows_v.at[1], ysem).wait()
    pltpu.sync_copy(myrows_v.at[0], my1_hbm.at[pl.ds(b0, BPW)])
    pltpu.sync_copy(myrows_v.at[1], my2_hbm.at[pl.ds(b0, BPW)])


_phase1 = functools.partial(
    pl.kernel,
    out_type=[
        jax.ShapeDtypeStruct((B, K1), jnp.float32),
        jax.ShapeDtypeStruct((B, K1), jnp.float32),
        jax.ShapeDtypeStruct((B, D), jnp.float32),
        jax.ShapeDtypeStruct((B, D), jnp.float32),
    ],
    mesh=plsc.VectorSubcoreMesh(core_axis_name="c", subcore_axis_name="s",
                                num_cores=NC, num_subcores=NS),
    scratch_types=[
        pltpu.VMEM((2, NCH, RCH), jnp.int32),
        pltpu.VMEM((2, D), jnp.float32),
        pltpu.VMEM((2, D), jnp.float32),
        pltpu.VMEM((2, RCH, D), jnp.float32),
        pltpu.VMEM((2, K1), jnp.float32),
        pltpu.VMEM((BPW,), jnp.int32),
        pltpu.VMEM((2, BPW, D), jnp.float32),
        pltpu.SemaphoreType.DMA,
        pltpu.SemaphoreType.DMA,
        pltpu.SemaphoreType.DMA,
        pltpu.SemaphoreType.DMA,
        pltpu.SemaphoreType.DMA,
        pltpu.SemaphoreType.DMA,
    ],
)(_sc_phase1)


def _tc_phase2_body(dA_ref, dB_ref, my1_ref, my2_ref, v1_ref, v2_ref,
                    outA_ref, outB_ref, upd1_ref, upd2_ref):
    eA = jnp.exp(dA_ref[...] / T)
    eB = jnp.exp(dB_ref[...] / T)
    scale = jnp.float32(N) / jnp.float32(B * K1)
    zA = jnp.sum(eA) * scale
    zB = jnp.sum(eB) * scale
    outA_ref[...] = eA / zA
    outB_ref[...] = eB / zB

    l1 = my1_ref[...] * MOM + v1_ref[...] * (1.0 - MOM)
    n1 = jnp.sqrt(jnp.sum(l1 * l1, axis=1, keepdims=True))
    upd1_ref[...] = l1 / n1
    l2 = my2_ref[...] * MOM + v2_ref[...] * (1.0 - MOM)
    n2 = jnp.sqrt(jnp.sum(l2 * l2, axis=1, keepdims=True))
    upd2_ref[...] = l2 / n2


_phase2 = pl.pallas_call(
    _tc_phase2_body,
    out_shape=[
        jax.ShapeDtypeStruct((B, K1), jnp.float32),
        jax.ShapeDtypeStruct((B, K1), jnp.float32),
        jax.ShapeDtypeStruct((B, D), jnp.float32),
        jax.ShapeDtypeStruct((B, D), jnp.float32),
    ],
)


def _tc_phase3_body(upd1_ref, upd2_ref, y_ref, win_ref, m1_ref, m2_ref,
                    o1_ref, o2_ref, sem1, sem2):
    def issue(i, _):
        wi = win_ref[i]
        yi = y_ref[i]
        pltpu.make_async_copy(upd1_ref.at[pl.ds(wi, 1)],
                              o1_ref.at[pl.ds(yi, 1)], sem1).start()
        pltpu.make_async_copy(upd2_ref.at[pl.ds(wi, 1)],
                              o2_ref.at[pl.ds(yi, 1)], sem2).start()
        return 0

    lax.fori_loop(0, B, issue, 0)
    pltpu.make_async_copy(upd1_ref, o1_ref.at[pl.ds(0, B)], sem1).wait()
    pltpu.make_async_copy(upd2_ref, o2_ref.at[pl.ds(0, B)], sem2).wait()


_phase3 = pl.pallas_call(
    _tc_phase3_body,
    in_specs=[
        pl.BlockSpec(memory_space=pltpu.VMEM),
        pl.BlockSpec(memory_space=pltpu.VMEM),
        pl.BlockSpec(memory_space=pltpu.SMEM),
        pl.BlockSpec(memory_space=pltpu.SMEM),
        pl.BlockSpec(memory_space=pltpu.MemorySpace.HBM),
        pl.BlockSpec(memory_space=pltpu.MemorySpace.HBM),
    ],
    out_specs=[
        pl.BlockSpec(memory_space=pltpu.MemorySpace.HBM),
        pl.BlockSpec(memory_space=pltpu.MemorySpace.HBM),
    ],
    out_shape=[
        jax.ShapeDtypeStruct((N, D), jnp.float32),
        jax.ShapeDtypeStruct((N, D), jnp.float32),
    ],
    input_output_aliases={4: 0, 5: 1},
    scratch_shapes=[pltpu.SemaphoreType.DMA, pltpu.SemaphoreType.DMA],
)


def kernel(v1, v2, memory_v1, memory_v2, y, idx):
    y = y.astype(jnp.int32)
    idx3 = idx.astype(jnp.int32).reshape(B, NCH, RCH)

    dotsA, dotsB, my1, my2 = _phase1(
        v1, v2, memory_v1, memory_v2, y, idx3)

    outA, outB, upd1, upd2 = _phase2(dotsA, dotsB, my1, my2, v1, v2)

    bi = jnp.arange(B, dtype=jnp.int32)
    winner = jnp.max(jnp.where(y[None, :] == y[:, None], bi[None, :], -1),
                     axis=1).astype(jnp.int32)

    new_m1, new_m2 = _phase3(upd1, upd2, y, winner, memory_v1, memory_v2)

    out_v1 = outB[..., None]
    out_v2 = outA[..., None]
    return (out_v1, out_v2, new_m1, new_m2)
